# baseline (device time: 36244 ns/iter reference)
import jax
import jax.numpy as jnp
from jax import lax
from jax.experimental import pallas as pl
from jax.experimental.pallas import tpu as pltpu

N_DEV = 8
B, SQ, DMODEL = 2, 128, 512
HL, DH = 4, 64
DL = HL * DH
SH = SQ // 2
ROWS = B * SQ
RS = ROWS // N_DEV

F32 = jnp.float32
BF16 = jnp.bfloat16

FIRST_HOP = {1: 1, 2: 1, 5: 1, 3: 3, 7: 3, 4: 4, 6: 4}
RELAY_FWD = {1: [2, 5], 3: [7], 4: [6]}
DEST_ORDER = [2, 5, 7, 6, 1, 3, 4]


def kernel(x, Wq, K_ext, V_ext, Wo):
    Kr = K_ext.reshape(B, 2, SH, N_DEV, DL).astype(BF16)
    Vr = V_ext.reshape(B, 2, SH, N_DEV, DL).astype(BF16)
    KVP = jnp.stack(
        [Kr.transpose(3, 1, 0, 2, 4), Vr.transpose(3, 1, 0, 2, 4)], axis=2
    )

    def body(
        x_ref, wq_ref, kvp_ref, wo_ref, out_ref,
        kv_recv, relay_buf, p_buf, rs_recv, ag_buf,
        kv_send_sems, kv_recv_sems, local_sem,
        relay_recv_sems, fwd_send_sems,
        rs_send_sems, rs_recv_sems,
        ag_send_sems, ag_recv_sems,
    ):
        my = lax.axis_index("i")

        barrier = pltpu.get_barrier_semaphore()

        @pl.when(my != 0)
        def _():
            pl.semaphore_signal(
                barrier, inc=1,
                device_id=(0,),
                device_id_type=pl.DeviceIdType.MESH,
            )

        kv_descs = []
        idx = 0
        for half in range(2):
            for dest in DEST_ORDER:
                hop = FIRST_HOP[dest]
                if hop == dest:
                    dst = kv_recv.at[half]
                    rsem = kv_recv_sems.at[half]
                else:
                    slot = RELAY_FWD[hop].index(dest)
                    dst = relay_buf.at[slot, half]
                    rsem = relay_recv_sems.at[slot * 2 + half]
                kv_descs.append(pltpu.make_async_remote_copy(
                    src_ref=kvp_ref.at[dest, half],
                    dst_ref=dst,
                    send_sem=kv_send_sems.at[idx],
                    recv_sem=rsem,
                    device_id=(hop,),
                    device_id_type=pl.DeviceIdType.MESH,
                ))
                idx += 1

        kv_rdescs = [
            pltpu.make_async_remote_copy(
                src_ref=kvp_ref.at[0, half],
                dst_ref=kv_recv.at[half],
                send_sem=kv_send_sems.at[0],
                recv_sem=kv_recv_sems.at[half],
                device_id=(0,),
                device_id_type=pl.DeviceIdType.MESH,
            )
            for half in range(2)
        ]

        loc_desc = pltpu.make_async_copy(
            kvp_ref.at[0], kv_recv, local_sem,
        )

        @pl.when(my == 0)
        def _():
            pl.semaphore_wait(barrier, N_DEV - 1)
            for d in kv_descs:
                d.start()
            loc_desc.start()

        relay_descs = {}
        for relay, dests in RELAY_FWD.items():
            recv_ds = [[], []]
            fwd_ds = [[], []]
            for half in range(2):
                for slot, dest in enumerate(dests):
                    recv_ds[half].append(pltpu.make_async_remote_copy(
                        src_ref=kvp_ref.at[0, half],
                        dst_ref=relay_buf.at[slot, half],
                        send_sem=kv_send_sems.at[0],
                        recv_sem=relay_recv_sems.at[slot * 2 + half],
                        device_id=(0,),
                        device_id_type=pl.DeviceIdType.MESH,
                    ))
                    fwd_ds[half].append(pltpu.make_async_remote_copy(
                        src_ref=relay_buf.at[slot, half],
                        dst_ref=kv_recv.at[half],
                        send_sem=fwd_send_sems.at[slot * 2 + half],
                        recv_sem=kv_recv_sems.at[half],
                        device_id=(dest,),
                        device_id_type=pl.DeviceIdType.MESH,
                    ))
            relay_descs[relay] = (recv_ds, fwd_ds)

            @pl.when(my == relay)
            def _(recv_ds=recv_ds, fwd_ds=fwd_ds):
                for rd, fd in zip(recv_ds[0], fwd_ds[0]):
                    rd.wait_recv()
                    fd.start()

        wq = wq_ref[...].astype(BF16)
        q = [
            jnp.dot(x_ref[b].astype(BF16), wq, preferred_element_type=F32
                    ).astype(BF16)
            for b in range(B)
        ]

        rs_send = []
        rs_targets = []
        for off in range(1, N_DEV):
            t = (my + off) % N_DEV
            rs_targets.append(t)
            rs_send.append(pltpu.make_async_remote_copy(
                src_ref=p_buf.at[pl.ds(t * RS, RS)],
                dst_ref=rs_recv.at[my],
                send_sem=rs_send_sems.at[off - 1],
                recv_sem=rs_recv_sems.at[my],
                device_id=(t,),
                device_id_type=pl.DeviceIdType.MESH,
            ))

        ch = lax.broadcasted_iota(jnp.int32, (SH, DL), 1)
        hmask = [(ch >= h * DH) & (ch < (h + 1) * DH) for h in range(HL)]
        wo = wo_ref[...].astype(BF16)

        def attend(qrows, chunks):
            ctx = jnp.zeros((SH, DL), F32)
            for h in range(HL):
                qh = jnp.where(hmask[h], qrows, 0)
                ss = [
                    lax.dot_general(
                        qh, kb, (((1,), (1,)), ((), ())),
                        preferred_element_type=F32,
                    ) * 0.125
                    for kb, _ in chunks
                ]
                m = ss[0].max(axis=-1, keepdims=True)
                for s in ss[1:]:
                    m = jnp.maximum(m, s.max(axis=-1, keepdims=True))
                es = [jnp.exp(s - m) for s in ss]
                den = es[0].sum(axis=-1, keepdims=True)
                for e in es[1:]:
                    den = den + e.sum(axis=-1, keepdims=True)
                ctxh = jnp.zeros((SH, DL), F32)
                for e, (_, vb) in zip(es, chunks):
                    ctxh += jnp.dot(
                        (e / den).astype(BF16), vb, preferred_element_type=F32)
                ctx += jnp.where(hmask[h], ctxh, 0)
            return ctx

        def store_and_send(b, half, ctx):
            pb = jnp.dot(ctx.astype(BF16), wo, preferred_element_type=F32)
            lo = b * SQ + half * SH
            p_buf[lo:lo + SH, :] = pb.astype(BF16)
            for d, t in zip(rs_send, rs_targets):
                @pl.when((t * RS >= lo) & (t * RS < lo + SH))
                def _(d=d):
                    d.start()

        @pl.when(my == 0)
        def _():
            loc_desc.wait()

        @pl.when(my != 0)
        def _():
            kv_rdescs[0].wait_recv()

        for b in range(B):
            ctx = attend(
                q[b][0:SH], [(kv_recv[0, 0, b], kv_recv[0, 1, b])])
            store_and_send(b, 0, ctx)

        for relay in RELAY_FWD:
            recv_ds, fwd_ds = relay_descs[relay]

            @pl.when(my == relay)
            def _(recv_ds=recv_ds, fwd_ds=fwd_ds):
                for rd, fd in zip(recv_ds[1], fwd_ds[1]):
                    rd.wait_recv()
                    fd.start()

        @pl.when(my != 0)
        def _():
            kv_rdescs[1].wait_recv()

        for b in range(B):
            ctx = attend(
                q[b][SH:SQ],
                [(kv_recv[0, 0, b], kv_recv[0, 1, b]),
                 (kv_recv[1, 0, b], kv_recv[1, 1, b])])
            store_and_send(b, 1, ctx)

        acc = p_buf[pl.ds(my * RS, RS)].astype(F32)
        for off in range(1, N_DEV):
            j = (my + off) % N_DEV
            rd = pltpu.make_async_remote_copy(
                src_ref=p_buf.at[pl.ds(0, RS)],
                dst_ref=rs_recv.at[j],
                send_sem=rs_send_sems.at[0],
                recv_sem=rs_recv_sems.at[j],
                device_id=(j,),
                device_id_type=pl.DeviceIdType.MESH,
            )
            rd.wait_recv()
            acc += rs_recv[j].astype(F32)

        ag_buf[my] = acc.astype(BF16)
        ag_send = []
        for off in range(1, N_DEV):
            t = (my + off) % N_DEV
            d = pltpu.make_async_remote_copy(
                src_ref=ag_buf.at[my],
                dst_ref=ag_buf.at[my],
                send_sem=ag_send_sems.at[off - 1],
                recv_sem=ag_recv_sems.at[my],
                device_id=(t,),
                device_id_type=pl.DeviceIdType.MESH,
            )
            d.start()
            ag_send.append(d)

        for off in range(1, N_DEV):
            j = (my + off) % N_DEV
            rd = pltpu.make_async_remote_copy(
                src_ref=ag_buf.at[0],
                dst_ref=ag_buf.at[j],
                send_sem=ag_send_sems.at[0],
                recv_sem=ag_recv_sems.at[j],
                device_id=(j,),
                device_id_type=pl.DeviceIdType.MESH,
            )
            rd.wait_recv()

        for s in range(N_DEV):
            b, r0 = divmod(s * RS, SQ)
            out_ref[b, r0:r0 + RS, :] = ag_buf[s].astype(F32)

        @pl.when(my == 0)
        def _():
            for d in kv_descs:
                d.wait_send()

        for relay in RELAY_FWD:
            recv_ds, fwd_ds = relay_descs[relay]

            @pl.when(my == relay)
            def _(fwd_ds=fwd_ds):
                for d in fwd_ds[0] + fwd_ds[1]:
                    d.wait_send()

        for d in rs_send:
            d.wait_send()
        for d in ag_send:
            d.wait_send()

    return pl.pallas_call(
        body,
        out_shape=jax.ShapeDtypeStruct((B, SQ, DMODEL), F32),
        in_specs=[pl.BlockSpec(memory_space=pltpu.VMEM)] * 4,
        out_specs=pl.BlockSpec(memory_space=pltpu.VMEM),
        scratch_shapes=[
            pltpu.VMEM((2, 2, B, SH, DL), BF16),
            pltpu.VMEM((2, 2, 2, B, SH, DL), BF16),
            pltpu.VMEM((ROWS, DMODEL), BF16),
            pltpu.VMEM((N_DEV, RS, DMODEL), BF16),
            pltpu.VMEM((N_DEV, RS, DMODEL), BF16),
            pltpu.SemaphoreType.DMA((2 * (N_DEV - 1),)),
            pltpu.SemaphoreType.DMA((2,)),
            pltpu.SemaphoreType.DMA,
            pltpu.SemaphoreType.DMA((4,)),
            pltpu.SemaphoreType.DMA((4,)),
            pltpu.SemaphoreType.DMA((N_DEV - 1,)),
            pltpu.SemaphoreType.DMA((N_DEV,)),
            pltpu.SemaphoreType.DMA((N_DEV - 1,)),
            pltpu.SemaphoreType.DMA((N_DEV,)),
        ],
        compiler_params=pltpu.CompilerParams(collective_id=0),
    )(x, Wq, KVP, Wo)


# device time: 28093 ns/iter; 1.2901x vs baseline; 1.2901x over previous
import jax
import jax.numpy as jnp
from jax import lax
from jax.experimental import pallas as pl
from jax.experimental.pallas import tpu as pltpu

N_DEV = 8
B, SQ, DMODEL = 2, 128, 512
HL, DH = 4, 64
DL = HL * DH
HG = 32 * 64
ROWS = B * SQ
RS = ROWS // N_DEV

F32 = jnp.float32
BF16 = jnp.bfloat16


def kernel(x, Wq, K_ext, V_ext, Wo):
    K2 = K_ext.reshape(B, SQ, HG).astype(BF16)
    V2 = V_ext.reshape(B, SQ, HG).astype(BF16)

    def body(
        x_ref, wq_ref, k_ref, v_ref, wo_ref, out_ref,
        kv_recv, relay_buf, p_buf, rs_recv, ag_buf,
        kv_send_sems, kv_recv_sems,
        relay_recv_sems, fwd_send_sems,
        rs_send_sems, rs_recv_sems,
        ag_send_sems, ag_recv_sems,
    ):
        my = lax.axis_index("i")

        barrier = pltpu.get_barrier_semaphore()

        @pl.when(my != 0)
        def _():
            pl.semaphore_signal(
                barrier, inc=1,
                device_id=(0,),
                device_id_type=pl.DeviceIdType.MESH,
            )

        FIRST_HOP = {1: 1, 2: 1, 5: 1, 3: 3, 7: 3, 4: 4, 6: 4}
        RELAY_FWD = {1: [2, 5], 3: [7], 4: [6]}
        SEND_ORDER = [2, 5, 7, 6, 1, 3, 4]
        kv_srcs = [k_ref, v_ref]
        kv_descs = []
        idx = 0
        for dest in SEND_ORDER:
            hop = FIRST_HOP[dest]
            for tv in range(2):
                if hop == dest:
                    dst, rsem = kv_recv.at[tv], kv_recv_sems.at[tv]
                else:
                    slot = RELAY_FWD[hop].index(dest)
                    dst = relay_buf.at[slot, tv]
                    rsem = relay_recv_sems.at[slot * 2 + tv]
                kv_descs.append(pltpu.make_async_remote_copy(
                    src_ref=kv_srcs[tv].at[:, :, pl.ds(dest * DL, DL)],
                    dst_ref=dst,
                    send_sem=kv_send_sems.at[idx],
                    recv_sem=rsem,
                    device_id=(hop,),
                    device_id_type=pl.DeviceIdType.MESH,
                ))
                idx += 1
        kv_rdescs = [
            pltpu.make_async_remote_copy(
                src_ref=kv_srcs[tv].at[:, :, pl.ds(0, DL)],
                dst_ref=kv_recv.at[tv],
                send_sem=kv_send_sems.at[0],
                recv_sem=kv_recv_sems.at[tv],
                device_id=(0,),
                device_id_type=pl.DeviceIdType.MESH,
            )
            for tv in range(2)
        ]

        @pl.when(my == 0)
        def _():
            pl.semaphore_wait(barrier, N_DEV - 1)
            for d in kv_descs:
                d.start()
            kv_recv[0] = k_ref[:, :, 0:DL]
            kv_recv[1] = v_ref[:, :, 0:DL]

        relay_fwd_descs = {}
        for relay, dests in RELAY_FWD.items():
            recv_ds, fwd_ds = [], []
            for slot, dest in enumerate(dests):
                for tv in range(2):
                    recv_ds.append(pltpu.make_async_remote_copy(
                        src_ref=kv_srcs[tv].at[:, :, pl.ds(0, DL)],
                        dst_ref=relay_buf.at[slot, tv],
                        send_sem=kv_send_sems.at[0],
                        recv_sem=relay_recv_sems.at[slot * 2 + tv],
                        device_id=(0,),
                        device_id_type=pl.DeviceIdType.MESH,
                    ))
                    fwd_ds.append(pltpu.make_async_remote_copy(
                        src_ref=relay_buf.at[slot, tv],
                        dst_ref=kv_recv.at[tv],
                        send_sem=fwd_send_sems.at[slot * 2 + tv],
                        recv_sem=kv_recv_sems.at[tv],
                        device_id=(dest,),
                        device_id_type=pl.DeviceIdType.MESH,
                    ))
            relay_fwd_descs[relay] = fwd_ds

            @pl.when(my == relay)
            def _(recv_ds=recv_ds, fwd_ds=fwd_ds):
                for rd, fd in zip(recv_ds, fwd_ds):
                    rd.wait_recv()
                    fd.start()

        wq = wq_ref[...].astype(BF16)
        q = [
            jnp.dot(x_ref[b].astype(BF16), wq, preferred_element_type=F32
                    ).astype(BF16)
            for b in range(B)
        ]

        @pl.when(my != 0)
        def _():
            for rd in kv_rdescs:
                rd.wait_recv()

        row = lax.broadcasted_iota(jnp.int32, (SQ, SQ), 0) // 64
        col = lax.broadcasted_iota(jnp.int32, (SQ, SQ), 1) // 64
        cmask = col <= row
        ch = lax.broadcasted_iota(jnp.int32, (SQ, DL), 1)
        wo = wo_ref[...].astype(BF16)

        rs_send = []
        rs_targets = []
        for off in range(1, N_DEV):
            t = (my + off) % N_DEV
            rs_targets.append(t)
            rs_send.append(pltpu.make_async_remote_copy(
                src_ref=p_buf.at[pl.ds(t * RS, RS)],
                dst_ref=rs_recv.at[my],
                send_sem=rs_send_sems.at[off - 1],
                recv_sem=rs_recv_sems.at[my],
                device_id=(t,),
                device_id_type=pl.DeviceIdType.MESH,
            ))

        for b in range(B):
            kb = kv_recv[0, b]
            vb = kv_recv[1, b]
            ctx = jnp.zeros((SQ, DL), F32)
            for h in range(HL):
                hm = (ch >= h * DH) & (ch < (h + 1) * DH)
                qh = jnp.where(hm, q[b], 0)
                s = lax.dot_general(
                    qh, kb, (((1,), (1,)), ((), ())),
                    preferred_element_type=F32,
                ) * 0.125
                s = jnp.where(cmask, s, -1e9)
                w = jnp.exp(s - jnp.max(s, axis=-1, keepdims=True))
                w = w / jnp.sum(w, axis=-1, keepdims=True)
                ctxh = jnp.dot(w.astype(BF16), vb, preferred_element_type=F32)
                ctx += jnp.where(hm, ctxh, 0)
            pb = jnp.dot(ctx.astype(BF16), wo, preferred_element_type=F32)
            p_buf[b * SQ:(b + 1) * SQ, :] = pb.astype(BF16)
            lo, hi = b * SQ, (b + 1) * SQ
            for d, t in zip(rs_send, rs_targets):
                @pl.when((t * RS >= lo) & (t * RS < hi))
                def _(d=d):
                    d.start()

        acc = p_buf[pl.ds(my * RS, RS)].astype(F32)
        for off in range(1, N_DEV):
            j = (my + off) % N_DEV
            rd = pltpu.make_async_remote_copy(
                src_ref=p_buf.at[pl.ds(0, RS)],
                dst_ref=rs_recv.at[j],
                send_sem=rs_send_sems.at[0],
                recv_sem=rs_recv_sems.at[j],
                device_id=(j,),
                device_id_type=pl.DeviceIdType.MESH,
            )
            rd.wait_recv()
            acc += rs_recv[j].astype(F32)

        ag_buf[my] = acc.astype(BF16)
        ag_send = []
        for off in range(1, N_DEV):
            t = (my + off) % N_DEV
            d = pltpu.make_async_remote_copy(
                src_ref=ag_buf.at[my],
                dst_ref=ag_buf.at[my],
                send_sem=ag_send_sems.at[off - 1],
                recv_sem=ag_recv_sems.at[my],
                device_id=(t,),
                device_id_type=pl.DeviceIdType.MESH,
            )
            d.start()
            ag_send.append(d)

        for off in range(1, N_DEV):
            j = (my + off) % N_DEV
            rd = pltpu.make_async_remote_copy(
                src_ref=ag_buf.at[0],
                dst_ref=ag_buf.at[j],
                send_sem=ag_send_sems.at[0],
                recv_sem=ag_recv_sems.at[j],
                device_id=(j,),
                device_id_type=pl.DeviceIdType.MESH,
            )
            rd.wait_recv()

        for s in range(N_DEV):
            b, r0 = divmod(s * RS, SQ)
            out_ref[b, r0:r0 + RS, :] = ag_buf[s].astype(F32)

        @pl.when(my == 0)
        def _():
            for d in kv_descs:
                d.wait_send()

        for relay, fwd_ds in relay_fwd_descs.items():
            @pl.when(my == relay)
            def _(fwd_ds=fwd_ds):
                for d in fwd_ds:
                    d.wait_send()

        for d in rs_send:
            d.wait_send()
        for d in ag_send:
            d.wait_send()

    return pl.pallas_call(
        body,
        out_shape=jax.ShapeDtypeStruct((B, SQ, DMODEL), F32),
        in_specs=[pl.BlockSpec(memory_space=pltpu.VMEM)] * 5,
        out_specs=pl.BlockSpec(memory_space=pltpu.VMEM),
        scratch_shapes=[
            pltpu.VMEM((2, B, SQ, DL), BF16),
            pltpu.VMEM((2, 2, B, SQ, DL), BF16),
            pltpu.VMEM((ROWS, DMODEL), BF16),
            pltpu.VMEM((N_DEV, RS, DMODEL), BF16),
            pltpu.VMEM((N_DEV, RS, DMODEL), BF16),
            pltpu.SemaphoreType.DMA((2 * (N_DEV - 1),)),
            pltpu.SemaphoreType.DMA((2,)),
            pltpu.SemaphoreType.DMA((4,)),
            pltpu.SemaphoreType.DMA((4,)),
            pltpu.SemaphoreType.DMA((N_DEV - 1,)),
            pltpu.SemaphoreType.DMA((N_DEV,)),
            pltpu.SemaphoreType.DMA((N_DEV - 1,)),
            pltpu.SemaphoreType.DMA((N_DEV,)),
        ],
        compiler_params=pltpu.CompilerParams(collective_id=0),
    )(x, Wq, K2, V2, Wo)
